# Initial kernel scaffold; baseline (speedup 1.0000x reference)
import jax
import jax.numpy as jnp
from jax.experimental import pallas as pl


def kernel(image, flow):
    raise NotImplementedError("write your pallas kernel here")


# speedup vs baseline: 1.2603x; 1.2603x over previous
import functools

import jax
import jax.numpy as jnp
from jax import lax
from jax.experimental import pallas as pl
from jax.experimental.pallas import tpu as pltpu
from jax.experimental.pallas import tpu_sc as plsc

B, H, W, C = 2, 512, 512, 96
P = B * H * W
LOG2_W = 9
LOG2_HW = 18

NC = 2
NS = 16
NW = NC * NS
PPW = P // NW
K = 128
NCHUNK = PPW // K
L = 16
NV = C // L


def _splat(vec, j):
    return vec.at[jnp.full((L,), j, jnp.int32)].get(mode="promise_in_bounds")


def _warp_body(img_hbm, fy_hbm, fx_hbm, out_hbm,
               fly_v, flx_v, idx_v, ax_v, ay_v, corners_v, out_v, sem):
    wid = lax.axis_index("s") * NC + lax.axis_index("c")
    lanes = lax.broadcasted_iota(jnp.int32, (L,), 0)

    def chunk_body(ci, _):
        base = wid * PPW + ci * K

        pltpu.sync_copy(fy_hbm.at[pl.ds(base, K)], fly_v)
        pltpu.sync_copy(fx_hbm.at[pl.ds(base, K)], flx_v)

        for g in range(K // L):
            rows = g * L + lanes
            p = base + rows
            x = p & (W - 1)
            y = (p >> LOG2_W) & (H - 1)
            bbase = (p >> LOG2_HW) << LOG2_HW

            sl16 = pl.ds(g * L, L)
            fl_y = fly_v[sl16]
            fl_x = flx_v[sl16]

            qy = jnp.clip(y.astype(jnp.float32) - fl_y, 0.0, float(H - 1))
            qx = jnp.clip(x.astype(jnp.float32) - fl_x, 0.0, float(W - 1))
            fy = jnp.minimum(qy.astype(jnp.int32), H - 2)
            fx = jnp.minimum(qx.astype(jnp.int32), W - 2)
            ay = jnp.clip(qy - fy.astype(jnp.float32), 0.0, 1.0)
            ax = jnp.clip(qx - fx.astype(jnp.float32), 0.0, 1.0)

            sl = pl.ds(g * L, L)
            rtl = bbase + (fy << LOG2_W) + fx
            idx_v[0, sl] = rtl
            idx_v[1, sl] = rtl + 1
            idx_v[2, sl] = rtl + W
            idx_v[3, sl] = rtl + W + 1
            ax_v[sl] = ax
            ay_v[sl] = ay

        cps = [pltpu.async_copy(img_hbm.at[idx_v.at[j]], corners_v.at[j], sem)
               for j in range(4)]
        for cp in cps:
            cp.wait()

        def blend(g, _):
            ax16 = ax_v[pl.ds(g * L, L)]
            ay16 = ay_v[pl.ds(g * L, L)]
            for j in range(L):
                i = g * L + j
                axs = _splat(ax16, j)
                ays = _splat(ay16, j)
                for v in range(NV):
                    csl = pl.ds(v * L, L)
                    tl = corners_v[0, i, csl]
                    tr = corners_v[1, i, csl]
                    bl = corners_v[2, i, csl]
                    br = corners_v[3, i, csl]
                    top = tl + axs * (tr - tl)
                    bot = bl + axs * (br - bl)
                    out_v[i, csl] = top + ays * (bot - top)
            return 0

        lax.fori_loop(0, K // L, blend, 0)

        pltpu.sync_copy(out_v, out_hbm.at[pl.ds(base, K)])
        return 0

    lax.fori_loop(0, NCHUNK, chunk_body, 0)


@jax.jit
def _warp(img_flat, flow_y, flow_x):
    f = pl.kernel(
        _warp_body,
        out_type=jax.ShapeDtypeStruct((P, C), jnp.float32),
        mesh=plsc.VectorSubcoreMesh(core_axis_name="c", subcore_axis_name="s"),
        compiler_params=pltpu.CompilerParams(use_tc_tiling_on_sc=False),
        scratch_types=[
            pltpu.VMEM((K,), jnp.float32),
            pltpu.VMEM((K,), jnp.float32),
            pltpu.VMEM((4, K), jnp.int32),
            pltpu.VMEM((K,), jnp.float32),
            pltpu.VMEM((K,), jnp.float32),
            pltpu.VMEM((4, K, C), jnp.float32),
            pltpu.VMEM((K, C), jnp.float32),
            pltpu.SemaphoreType.DMA,
        ],
    )
    return f(img_flat, flow_y, flow_x)


def kernel(image, flow):
    img_flat = image.reshape(P, C)
    flow_y = flow[..., 0].reshape(P)
    flow_x = flow[..., 1].reshape(P)
    out = _warp(img_flat, flow_y, flow_x)
    return out.reshape(B, H, W, C)
